# baseline (speedup 1.0000x reference)
import functools

import jax
import jax.numpy as jnp
from jax import lax
from jax.experimental import pallas as pl
from jax.experimental.pallas import tpu as pltpu
from jax.experimental.pallas import tpu_sc as plsc

_G = 256
_EPS = 1e-9
_L = 16
_NC, _NS = 2, 16
_C = 400
_CA = 400


def _sc_mesh():
  return plsc.VectorSubcoreMesh(
      core_axis_name="c", subcore_axis_name="s",
      num_cores=_NC, num_subcores=_NS)


def _stats(x, batch, interpret=False):
  n, d = x.shape
  dh = d // _NC
  nfh = dh // _L
  n_chunks = n // _C
  assert n_chunks * _C == n

  @functools.partial(
      pl.kernel,
      out_type=[
          jax.ShapeDtypeStruct((_NC, _NS, _G, dh), jnp.float32),
          jax.ShapeDtypeStruct((_NC, _NS, _G, dh), jnp.float32),
          jax.ShapeDtypeStruct((_NC, _NS, _G, _L), jnp.float32),
      ],
      mesh=_sc_mesh(),
      compiler_params=pltpu.CompilerParams(use_tc_tiling_on_sc=False),
      scratch_types=[
          pltpu.VMEM((2, _C, dh), jnp.float32),
          pltpu.VMEM((_C,), jnp.int32),
          pltpu.VMEM((_C,), jnp.int32),
          pltpu.VMEM((_G, dh), jnp.float32),
          pltpu.VMEM((_G, dh), jnp.float32),
          pltpu.VMEM((_G, _L), jnp.float32),
          pltpu.SemaphoreType.DMA,
          pltpu.SemaphoreType.DMA,
      ],
      interpret=interpret,
  )
  def k(x_hbm, b_hbm, sum_hbm, sq_hbm, cnt_hbm, xv, iv0, iv1, sumv, sqv,
        cntv, sem0, sem1):
    cid = lax.axis_index("c")
    sid = lax.axis_index("s")
    col0 = cid * dh
    zeros = jnp.zeros((_L,), jnp.float32)
    ones = jnp.ones((_L,), jnp.float32)

    lo = (n_chunks * sid) // _NS
    hi = (n_chunks * (sid + 1)) // _NS

    def in_start(c, b):
      rows = pl.ds(c * _C, _C)

      @pl.when(b == 0)
      def _():
        pltpu.async_copy(x_hbm.at[rows, pl.ds(col0, dh)], xv.at[0], sem0)
        pltpu.async_copy(b_hbm.at[rows], iv0, sem0)

      @pl.when(b == 1)
      def _():
        pltpu.async_copy(x_hbm.at[rows, pl.ds(col0, dh)], xv.at[1], sem1)
        pltpu.async_copy(b_hbm.at[rows], iv1, sem1)

    def in_wait(b):
      rows = pl.ds(0, _C)

      @pl.when(b == 0)
      def _():
        pltpu.make_async_copy(
            x_hbm.at[rows, pl.ds(0, dh)], xv.at[0], sem0).wait()
        pltpu.make_async_copy(b_hbm.at[rows], iv0, sem0).wait()

      @pl.when(b == 1)
      def _():
        pltpu.make_async_copy(
            x_hbm.at[rows, pl.ds(0, dh)], xv.at[1], sem1).wait()
        pltpu.make_async_copy(b_hbm.at[rows], iv1, sem1).wait()

    def zero_body(g, carry):
      for f in range(nfh):
        s = pl.ds(f * _L, _L)
        sumv[g, s] = zeros
        sqv[g, s] = zeros
      cntv[g, :] = zeros
      return carry

    in_start(lo, 0)
    lax.fori_loop(0, _G, zero_body, 0)

    def chunk_body(c, carry):
      b = lax.rem(c - lo, 2)

      @pl.when(c + 1 < hi)
      def _():
        in_start(c + 1, 1 - b)

      in_wait(b)

      def grp_body(q, rc):
        gvec = jnp.where(b == 0, iv0[pl.ds(q * _L, _L)],
                         iv1[pl.ds(q * _L, _L)])
        g0 = gvec[0]
        g15 = gvec[_L - 1]

        @pl.when(g0 == g15)
        def _fast():
          accs = []
          accq = []
          for f in range(nfh):
            s = pl.ds(f * _L, _L)
            v = xv[b, q * _L, s]
            accs.append(v)
            accq.append(v * v)
          for j in range(1, _L):
            r = q * _L + j
            for f in range(nfh):
              s = pl.ds(f * _L, _L)
              v = xv[b, r, s]
              accs[f] = accs[f] + v
              accq[f] = accq[f] + v * v
          for f in range(nfh):
            s = pl.ds(f * _L, _L)
            plsc.addupdate(sumv.at[g0, s], accs[f])
            plsc.addupdate(sqv.at[g0, s], accq[f])
          plsc.addupdate(cntv.at[g0, :], ones * float(_L))

        @pl.when(g0 != g15)
        def _slow():
          for j in range(_L):
            g = gvec[j]
            r = q * _L + j
            for f in range(nfh):
              s = pl.ds(f * _L, _L)
              v = xv[b, r, s]
              plsc.addupdate(sumv.at[g, s], v)
              plsc.addupdate(sqv.at[g, s], v * v)
            plsc.addupdate(cntv.at[g, :], ones)

        return rc

      lax.fori_loop(0, _C // _L, grp_body, 0)
      return carry

    lax.fori_loop(lo, hi, chunk_body, 0)
    pltpu.sync_copy(sumv, sum_hbm.at[cid, sid])
    pltpu.sync_copy(sqv, sq_hbm.at[cid, sid])
    pltpu.sync_copy(cntv, cnt_hbm.at[cid, sid])

  return k(x, batch)


def _combine(sum_p, sq_p, cnt_p, alpha, weight, bias, interpret=False):
  dh = sum_p.shape[-1]

  def k(sum_ref, sq_ref, cnt_ref, a_ref, w_ref, b_ref, scale_ref, shift_ref):
    sums_h = jnp.sum(sum_ref[...], axis=1)
    sqs_h = jnp.sum(sq_ref[...], axis=1)
    sums = jnp.concatenate([sums_h[0], sums_h[1]], axis=-1)
    sqs = jnp.concatenate([sqs_h[0], sqs_h[1]], axis=-1)
    cnt = jnp.sum(cnt_ref[...], axis=(0, 1))[:, 0:1] * (1.0 / _NC)
    cnt = jnp.maximum(cnt, 1.0)
    mu = sums / cnt
    m2 = sqs / cnt
    a = a_ref[...]
    w = w_ref[...]
    b = b_ref[...]
    sig2 = m2 - (2.0 * a - a * a) * mu * mu
    sig2 = jnp.maximum(sig2, 0.0) + _EPS
    rstd = lax.rsqrt(sig2)
    scale = w * rstd
    shift = b - w * a * mu * rstd
    scale_ref[...] = jnp.stack([scale[:, :dh], scale[:, dh:]])
    shift_ref[...] = jnp.stack([shift[:, :dh], shift[:, dh:]])

  return pl.pallas_call(
      k,
      out_shape=[
          jax.ShapeDtypeStruct((_NC, _G, dh), jnp.float32),
          jax.ShapeDtypeStruct((_NC, _G, dh), jnp.float32),
      ],
      interpret=interpret,
  )(sum_p, sq_p, cnt_p, alpha, weight, bias)


def _apply(x, batch, scale, shift, interpret=False):
  n, d = x.shape
  dh = d // _NC
  nfh = dh // _L
  n_chunks = n // _CA
  assert n_chunks * _CA == n

  @functools.partial(
      pl.kernel,
      out_type=jax.ShapeDtypeStruct((n, d), jnp.float32),
      mesh=_sc_mesh(),
      compiler_params=pltpu.CompilerParams(use_tc_tiling_on_sc=False),
      scratch_types=[
          pltpu.VMEM((3, _CA, dh), jnp.float32),
          pltpu.VMEM((_CA,), jnp.int32),
          pltpu.VMEM((_CA,), jnp.int32),
          pltpu.VMEM((_CA,), jnp.int32),
          pltpu.VMEM((_G, dh), jnp.float32),
          pltpu.VMEM((_G, dh), jnp.float32),
          pltpu.SemaphoreType.DMA,
          pltpu.SemaphoreType.DMA,
          pltpu.SemaphoreType.DMA,
          pltpu.SemaphoreType.DMA,
          pltpu.SemaphoreType.DMA,
          pltpu.SemaphoreType.DMA,
      ],
      interpret=interpret,
  )
  def k(x_hbm, b_hbm, sc_hbm, sh_hbm, out_hbm, xv, iv0, iv1, iv2, scv, shv,
        semi0, semi1, semi2, semo0, semo1, semo2):
    cid = lax.axis_index("c")
    sid = lax.axis_index("s")
    col0 = cid * dh

    lo = (n_chunks * sid) // _NS
    hi = (n_chunks * (sid + 1)) // _NS

    def in_start(c, b):
      rows = pl.ds(c * _CA, _CA)

      @pl.when(b == 0)
      def _():
        pltpu.async_copy(x_hbm.at[rows, pl.ds(col0, dh)], xv.at[0], semi0)
        pltpu.async_copy(b_hbm.at[rows], iv0, semi0)

      @pl.when(b == 1)
      def _():
        pltpu.async_copy(x_hbm.at[rows, pl.ds(col0, dh)], xv.at[1], semi1)
        pltpu.async_copy(b_hbm.at[rows], iv1, semi1)

      @pl.when(b == 2)
      def _():
        pltpu.async_copy(x_hbm.at[rows, pl.ds(col0, dh)], xv.at[2], semi2)
        pltpu.async_copy(b_hbm.at[rows], iv2, semi2)

    def in_wait(b):
      rows = pl.ds(0, _CA)

      @pl.when(b == 0)
      def _():
        pltpu.make_async_copy(
            x_hbm.at[rows, pl.ds(0, dh)], xv.at[0], semi0).wait()
        pltpu.make_async_copy(b_hbm.at[rows], iv0, semi0).wait()

      @pl.when(b == 1)
      def _():
        pltpu.make_async_copy(
            x_hbm.at[rows, pl.ds(0, dh)], xv.at[1], semi1).wait()
        pltpu.make_async_copy(b_hbm.at[rows], iv1, semi1).wait()

      @pl.when(b == 2)
      def _():
        pltpu.make_async_copy(
            x_hbm.at[rows, pl.ds(0, dh)], xv.at[2], semi2).wait()
        pltpu.make_async_copy(b_hbm.at[rows], iv2, semi2).wait()

    def out_start(c, b):
      rows = pl.ds(c * _CA, _CA)

      @pl.when(b == 0)
      def _():
        pltpu.async_copy(xv.at[0], out_hbm.at[rows, pl.ds(col0, dh)], semo0)

      @pl.when(b == 1)
      def _():
        pltpu.async_copy(xv.at[1], out_hbm.at[rows, pl.ds(col0, dh)], semo1)

      @pl.when(b == 2)
      def _():
        pltpu.async_copy(xv.at[2], out_hbm.at[rows, pl.ds(col0, dh)], semo2)

    def out_wait(b):
      rows = pl.ds(0, _CA)

      @pl.when(b == 0)
      def _():
        pltpu.make_async_copy(
            xv.at[0], out_hbm.at[rows, pl.ds(0, dh)], semo0).wait()

      @pl.when(b == 1)
      def _():
        pltpu.make_async_copy(
            xv.at[1], out_hbm.at[rows, pl.ds(0, dh)], semo1).wait()

      @pl.when(b == 2)
      def _():
        pltpu.make_async_copy(
            xv.at[2], out_hbm.at[rows, pl.ds(0, dh)], semo2).wait()

    in_start(lo, 0)
    pltpu.sync_copy(sc_hbm.at[cid], scv)
    pltpu.sync_copy(sh_hbm.at[cid], shv)

    @pl.when(lo + 1 < hi)
    def _():
      in_start(lo + 1, 1)

    def chunk_body(c, carry):
      b = lax.rem(c - lo, 3)

      in_wait(b)

      def grp_body(q, rc):
        gvec = jnp.where(
            b == 0, iv0[pl.ds(q * _L, _L)],
            jnp.where(b == 1, iv1[pl.ds(q * _L, _L)],
                      iv2[pl.ds(q * _L, _L)]))
        g0 = gvec[0]
        g15 = gvec[_L - 1]

        @pl.when(g0 == g15)
        def _fast():
          scr = []
          shr = []
          for f in range(nfh):
            s = pl.ds(f * _L, _L)
            scr.append(scv[g0, s])
            shr.append(shv[g0, s])
          for j in range(_L):
            r = q * _L + j
            for f in range(nfh):
              s = pl.ds(f * _L, _L)
              xv[b, r, s] = xv[b, r, s] * scr[f] + shr[f]

        @pl.when(g0 != g15)
        def _slow():
          for j in range(_L):
            g = gvec[j]
            r = q * _L + j
            for f in range(nfh):
              s = pl.ds(f * _L, _L)
              xv[b, r, s] = xv[b, r, s] * scv[g, s] + shv[g, s]

        return rc

      lax.fori_loop(0, _CA // _L, grp_body, 0)
      out_start(c, b)

      @pl.when(c + 2 < hi)
      def _():
        b2 = lax.rem(c + 2 - lo, 3)

        @pl.when(c - 1 >= lo)
        def _():
          out_wait(b2)

        in_start(c + 2, b2)

      return carry

    lax.fori_loop(lo, hi, chunk_body, 0)

    def drain(i, carry):
      @pl.when(i < hi)
      def _():
        out_wait(lax.rem(i - lo, 3))
      return carry

    lax.fori_loop(jnp.maximum(lo, hi - 3), hi, drain, 0)

  return k(x, batch, scale, shift)


def kernel(x, batch, alpha, weight, bias):
  batch = batch.astype(jnp.int32)
  sum_p, sq_p, cnt_p = _stats(x, batch)
  scale, shift = _combine(
      sum_p, sq_p, cnt_p,
      alpha.reshape(1, -1), weight.reshape(1, -1), bias.reshape(1, -1))
  return _apply(x, batch, scale, shift)


# speedup vs baseline: 1.9173x; 1.9173x over previous
import functools

import jax
import jax.numpy as jnp
from jax import lax
from jax.experimental import pallas as pl
from jax.experimental.pallas import tpu as pltpu
from jax.experimental.pallas import tpu_sc as plsc

_G = 256
_EPS = 1e-9
_L = 16
_NC, _NS = 2, 16
_GS = _G // _NS
_C = 160


def _sc_mesh():
  return plsc.VectorSubcoreMesh(
      core_axis_name="c", subcore_axis_name="s",
      num_cores=_NC, num_subcores=_NS)


def _nr_rsqrt(v):
  i = plsc.bitcast(v, jnp.int32)
  i = 0x5F3759DF - lax.shift_right_logical(i, 1)
  y = plsc.bitcast(i, jnp.float32)
  for _ in range(3):
    y = y * (1.5 - 0.5 * v * y * y)
  return y


def _fused(x, batch, alpha, weight, bias, interpret=False):
  n, d = x.shape
  dh = d // _NC
  nfh = dh // _L
  n_chunks = n // _C
  assert n_chunks * _C == n
  assert _GS * _NS == _G

  @functools.partial(
      pl.kernel,
      out_type=jax.ShapeDtypeStruct((n, d), jnp.float32),
      mesh=_sc_mesh(),
      compiler_params=pltpu.CompilerParams(use_tc_tiling_on_sc=False, needs_layout_passes=False),
      scratch_types=[
          pltpu.VMEM((2, _C, dh), jnp.float32),
          pltpu.VMEM((2, _C, dh), jnp.float32),
          pltpu.VMEM((_C,), jnp.int32),
          pltpu.VMEM((_C,), jnp.int32),
          pltpu.VMEM((_G, dh), jnp.float32),
          pltpu.VMEM((_G, dh), jnp.float32),
          pltpu.VMEM((_G, _L), jnp.float32),
          pltpu.VMEM((_NS, _GS, dh), jnp.float32),
          pltpu.VMEM((_NS, _GS, _L), jnp.float32),
          pltpu.VMEM((_GS, dh), jnp.float32),
          pltpu.VMEM((_GS, dh), jnp.float32),
          pltpu.VMEM((_GS, dh), jnp.float32),
          pltpu.VMEM((_GS, dh), jnp.float32),
          pltpu.VMEM((1, dh), jnp.float32),
          pltpu.VMEM((1, dh), jnp.float32),
          pltpu.VMEM((dh,), jnp.float32),
          pltpu.VMEM((dh,), jnp.float32),
          pltpu.VMEM((dh,), jnp.float32),
          pltpu.VMEM_SHARED((_NS, _G, dh), jnp.float32),
          pltpu.VMEM_SHARED((_NS, _G, _L), jnp.float32),
          pltpu.VMEM_SHARED((_G, dh), jnp.float32),
          pltpu.VMEM_SHARED((_G, dh), jnp.float32),
          pltpu.SemaphoreType.DMA,
          pltpu.SemaphoreType.DMA,
          pltpu.SemaphoreType.DMA,
          pltpu.SemaphoreType.DMA,
          pltpu.SemaphoreType.DMA,
      ],
      interpret=interpret,
  )
  def k(x_hbm, b_hbm, a_hbm, w_hbm, bias_hbm, out_hbm,
        xv, ov, iv0, iv1, sumv, sqv, cntv, red, red_c,
        acc_s, acc_q, slc_sc, slc_sh, stag_sc, stag_sh, av, wv, bv,
        spm_p, spm_c, spm_sc, spm_sh,
        semi0, semi1, semo0, semo1, semr):
    cid = lax.axis_index("c")
    sid = lax.axis_index("s")
    col0 = cid * dh
    zeros = jnp.zeros((_L,), jnp.float32)
    ones = jnp.ones((_L,), jnp.float32)

    lo = (n_chunks * sid) // _NS
    hi = (n_chunks * (sid + 1)) // _NS

    def in_start(c, b):
      rows = pl.ds(c * _C, _C)

      @pl.when(b == 0)
      def _():
        pltpu.async_copy(x_hbm.at[rows, pl.ds(col0, dh)], xv.at[0], semi0)
        pltpu.async_copy(b_hbm.at[rows], iv0, semi0)

      @pl.when(b == 1)
      def _():
        pltpu.async_copy(x_hbm.at[rows, pl.ds(col0, dh)], xv.at[1], semi1)
        pltpu.async_copy(b_hbm.at[rows], iv1, semi1)

    def in_wait(b):
      rows = pl.ds(0, _C)

      @pl.when(b == 0)
      def _():
        pltpu.make_async_copy(
            x_hbm.at[rows, pl.ds(0, dh)], xv.at[0], semi0).wait()
        pltpu.make_async_copy(b_hbm.at[rows], iv0, semi0).wait()

      @pl.when(b == 1)
      def _():
        pltpu.make_async_copy(
            x_hbm.at[rows, pl.ds(0, dh)], xv.at[1], semi1).wait()
        pltpu.make_async_copy(b_hbm.at[rows], iv1, semi1).wait()

    def out_start(c, b):
      rows = pl.ds(c * _C, _C)

      @pl.when(b == 0)
      def _():
        pltpu.async_copy(ov.at[0], out_hbm.at[rows, pl.ds(col0, dh)], semo0)

      @pl.when(b == 1)
      def _():
        pltpu.async_copy(ov.at[1], out_hbm.at[rows, pl.ds(col0, dh)], semo1)

    def out_wait(b):
      rows = pl.ds(0, _C)

      @pl.when(b == 0)
      def _():
        pltpu.make_async_copy(
            ov.at[0], out_hbm.at[rows, pl.ds(0, dh)], semo0).wait()

      @pl.when(b == 1)
      def _():
        pltpu.make_async_copy(
            ov.at[1], out_hbm.at[rows, pl.ds(0, dh)], semo1).wait()

    def gvec_of(b, q):
      return jnp.where(b == 0, iv0[pl.ds(q * _L, _L)],
                       iv1[pl.ds(q * _L, _L)])

    in_start(lo, 0)
    pltpu.sync_copy(a_hbm.at[pl.ds(col0, dh)], av)
    pltpu.sync_copy(w_hbm.at[pl.ds(col0, dh)], wv)
    pltpu.sync_copy(bias_hbm.at[pl.ds(col0, dh)], bv)

    def zero_body(g, carry):
      for f in range(nfh):
        s = pl.ds(f * _L, _L)
        sumv[g, s] = zeros
        sqv[g, s] = zeros
      cntv[g, :] = zeros
      return carry

    lax.fori_loop(0, _G, zero_body, 0)

    def stats_chunk(c, carry):
      b = lax.rem(c - lo, 2)

      @pl.when(c + 1 < hi)
      def _():
        in_start(c + 1, 1 - b)

      in_wait(b)

      def grp_body(q, rc):
        gvec = gvec_of(b, q)
        g0 = gvec[0]
        g15 = gvec[_L - 1]

        @pl.when(g0 == g15)
        def _fast():
          accs = []
          accq = []
          for f in range(nfh):
            s = pl.ds(f * _L, _L)
            v = xv[b, q * _L, s]
            accs.append(v)
            accq.append(v * v)
          for j in range(1, _L):
            r = q * _L + j
            for f in range(nfh):
              s = pl.ds(f * _L, _L)
              v = xv[b, r, s]
              accs[f] = accs[f] + v
              accq[f] = accq[f] + v * v
          for f in range(nfh):
            s = pl.ds(f * _L, _L)
            plsc.addupdate(sumv.at[g0, s], accs[f])
            plsc.addupdate(sqv.at[g0, s], accq[f])
          plsc.addupdate(cntv.at[g0, :], ones * float(_L))

        @pl.when(g0 != g15)
        def _slow():
          for j in range(_L):
            g = gvec[j]
            r = q * _L + j
            for f in range(nfh):
              s = pl.ds(f * _L, _L)
              v = xv[b, r, s]
              plsc.addupdate(sumv.at[g, s], v)
              plsc.addupdate(sqv.at[g, s], v * v)
            plsc.addupdate(cntv.at[g, :], ones)

        return rc

      lax.fori_loop(0, _C // _L, grp_body, 0)
      return carry

    lax.fori_loop(lo, hi, stats_chunk, 0)

    g0s = sid * _GS

    pltpu.sync_copy(sumv, spm_p.at[sid])
    pltpu.sync_copy(cntv, spm_c.at[sid])
    plsc.subcore_barrier()

    handles = []
    for t in range(_NS):
      handles.append(pltpu.async_copy(
          spm_c.at[t, pl.ds(g0s, _GS)], red_c.at[t], semr))
    for h in handles:
      h.wait()

    handles = []
    for t in range(_NS):
      handles.append(pltpu.async_copy(
          spm_p.at[t, pl.ds(g0s, _GS)], red.at[t], semr))
    for h in handles:
      h.wait()

    def red_s_body(gi, carry):
      for f in range(nfh):
        s = pl.ds(f * _L, _L)
        acc = red[0, gi, s]
        for t in range(1, _NS):
          acc = acc + red[t, gi, s]
        acc_s[gi, s] = acc
      return carry

    lax.fori_loop(0, _GS, red_s_body, 0)

    plsc.subcore_barrier()
    pltpu.sync_copy(sqv, spm_p.at[sid])
    plsc.subcore_barrier()

    handles = []
    for t in range(_NS):
      handles.append(pltpu.async_copy(
          spm_p.at[t, pl.ds(g0s, _GS)], red.at[t], semr))
    for h in handles:
      h.wait()

    def red_q_body(gi, carry):
      for f in range(nfh):
        s = pl.ds(f * _L, _L)
        acc = red[0, gi, s]
        for t in range(1, _NS):
          acc = acc + red[t, gi, s]
        acc_q[gi, s] = acc
      return carry

    lax.fori_loop(0, _GS, red_q_body, 0)

    def scale_body(gi, carry):
      cvec = red_c[0, gi, :]
      for t in range(1, _NS):
        cvec = cvec + red_c[t, gi, :]
      cnt = jnp.maximum(cvec, 1.0)
      rcnt = 1.0 / cnt
      for f in range(nfh):
        s = pl.ds(f * _L, _L)
        mu = acc_s[gi, s] * rcnt
        m2 = acc_q[gi, s] * rcnt
        a = av[s]
        w = wv[s]
        bb = bv[s]
        sig2 = m2 - (2.0 * a - a * a) * mu * mu
        sig2 = jnp.maximum(sig2, 0.0) + _EPS
        y = _nr_rsqrt(sig2)
        slc_sc[gi, s] = w * y
        slc_sh[gi, s] = bb - w * a * mu * y
      return carry

    lax.fori_loop(0, _GS, scale_body, 0)

    pltpu.sync_copy(slc_sc, spm_sc.at[pl.ds(g0s, _GS)])
    pltpu.sync_copy(slc_sh, spm_sh.at[pl.ds(g0s, _GS)])
    plsc.subcore_barrier()

    def fetch(g):
      pltpu.sync_copy(spm_sc.at[pl.ds(g, 1)], stag_sc)
      pltpu.sync_copy(spm_sh.at[pl.ds(g, 1)], stag_sh)

    in_start(lo, 0)

    def apply_chunk(c, cur):
      b = lax.rem(c - lo, 2)

      @pl.when(c + 1 < hi)
      def _():
        in_start(c + 1, 1 - b)

      in_wait(b)

      @pl.when(c - 2 >= lo)
      def _():
        out_wait(b)

      def grp_body(q, gcur):
        gvec = gvec_of(b, q)
        g0 = gvec[0]
        g15 = gvec[_L - 1]

        @pl.when(g0 != gcur)
        def _():
          fetch(g0)

        @pl.when(g0 == g15)
        def _fast():
          scr = []
          shr = []
          for f in range(nfh):
            s = pl.ds(f * _L, _L)
            scr.append(stag_sc[0, s])
            shr.append(stag_sh[0, s])
          for j0 in range(0, _L, 4):
            vals = []
            for j in range(4):
              r = q * _L + j0 + j
              for f in range(nfh):
                vals.append(xv[b, r, pl.ds(f * _L, _L)])
            res = []
            for j in range(4):
              for f in range(nfh):
                res.append(vals[j * nfh + f] * scr[f] + shr[f])
            for j in range(4):
              r = q * _L + j0 + j
              for f in range(nfh):
                ov[b, r, pl.ds(f * _L, _L)] = res[j * nfh + f]

        @pl.when(g0 != g15)
        def _slow():
          prev = g0
          for j in range(_L):
            g = gvec[j]
            r = q * _L + j
            if j > 0:
              pred = g != prev

              @pl.when(pred)
              def _():
                fetch(g)

            for f in range(nfh):
              s = pl.ds(f * _L, _L)
              ov[b, r, s] = xv[b, r, s] * stag_sc[0, s] + stag_sh[0, s]
            prev = g

        return g15

      gcur = lax.fori_loop(0, _C // _L, grp_body, cur)
      out_start(c, b)
      return gcur

    lax.fori_loop(lo, hi, apply_chunk, jnp.int32(-1))

    def drain(i, carry):
      @pl.when(i >= lo)
      def _():
        out_wait(lax.rem(i - lo, 2))
      return carry

    lax.fori_loop(hi - 2, hi, drain, 0)

  return k(x, batch, alpha, weight, bias)


def kernel(x, batch, alpha, weight, bias):
  batch = batch.astype(jnp.int32)
  return _fused(x, batch, alpha, weight, bias)


# speedup vs baseline: 2.0542x; 1.0714x over previous
import functools

import jax
import jax.numpy as jnp
from jax import lax
from jax.experimental import pallas as pl
from jax.experimental.pallas import tpu as pltpu
from jax.experimental.pallas import tpu_sc as plsc

_G = 256
_EPS = 1e-9
_L = 16
_NC, _NS = 2, 16
_GS = _G // _NS
_C = 160


def _sc_mesh():
  return plsc.VectorSubcoreMesh(
      core_axis_name="c", subcore_axis_name="s",
      num_cores=_NC, num_subcores=_NS)


def _nr_rsqrt(v):
  i = plsc.bitcast(v, jnp.int32)
  i = 0x5F3759DF - lax.shift_right_logical(i, 1)
  y = plsc.bitcast(i, jnp.float32)
  for _ in range(3):
    y = y * (1.5 - 0.5 * v * y * y)
  return y


def _fused(x, batch, alpha, weight, bias, interpret=False):
  n, d = x.shape
  dh = d // _NC
  nfh = dh // _L
  n_chunks = n // _C
  assert n_chunks * _C == n
  assert _GS * _NS == _G

  @functools.partial(
      pl.kernel,
      out_type=jax.ShapeDtypeStruct((n, d), jnp.float32),
      mesh=_sc_mesh(),
      compiler_params=pltpu.CompilerParams(use_tc_tiling_on_sc=False, needs_layout_passes=False),
      scratch_types=[
          pltpu.VMEM((2, _C, dh), jnp.float32),
          pltpu.VMEM((2, _C, dh), jnp.float32),
          pltpu.VMEM((_C,), jnp.int32),
          pltpu.VMEM((_C,), jnp.int32),
          pltpu.VMEM((_G, dh), jnp.float32),
          pltpu.VMEM((_G, dh), jnp.float32),
          pltpu.VMEM((_G, _L), jnp.float32),
          pltpu.VMEM((_NS, _GS, dh), jnp.float32),
          pltpu.VMEM((_NS, _GS, _L), jnp.float32),
          pltpu.VMEM((_GS, dh), jnp.float32),
          pltpu.VMEM((_GS, dh), jnp.float32),
          pltpu.VMEM((_GS, dh), jnp.float32),
          pltpu.VMEM((_GS, dh), jnp.float32),
          pltpu.VMEM((1, dh), jnp.float32),
          pltpu.VMEM((1, dh), jnp.float32),
          pltpu.VMEM((dh,), jnp.float32),
          pltpu.VMEM((dh,), jnp.float32),
          pltpu.VMEM((dh,), jnp.float32),
          pltpu.VMEM_SHARED((_NS, _G, dh), jnp.float32),
          pltpu.VMEM_SHARED((_NS, _G, _L), jnp.float32),
          pltpu.VMEM_SHARED((_G, dh), jnp.float32),
          pltpu.VMEM_SHARED((_G, dh), jnp.float32),
          pltpu.SemaphoreType.DMA,
          pltpu.SemaphoreType.DMA,
          pltpu.SemaphoreType.DMA,
          pltpu.SemaphoreType.DMA,
          pltpu.SemaphoreType.DMA,
      ],
      interpret=interpret,
  )
  def k(x_hbm, b_hbm, a_hbm, w_hbm, bias_hbm, out_hbm,
        xv, ov, iv0, iv1, sumv, sqv, cntv, red, red_c,
        acc_s, acc_q, slc_sc, slc_sh, stag_sc, stag_sh, av, wv, bv,
        spm_p, spm_c, spm_sc, spm_sh,
        semi0, semi1, semo0, semo1, semr):
    cid = lax.axis_index("c")
    sid = lax.axis_index("s")
    col0 = cid * dh
    zeros = jnp.zeros((_L,), jnp.float32)
    ones = jnp.ones((_L,), jnp.float32)

    lo = (n_chunks * sid) // _NS
    hi = (n_chunks * (sid + 1)) // _NS

    def in_start(c, b):
      rows = pl.ds(c * _C, _C)

      @pl.when(b == 0)
      def _():
        pltpu.async_copy(x_hbm.at[rows, pl.ds(col0, dh)], xv.at[0], semi0)
        pltpu.async_copy(b_hbm.at[rows], iv0, semi0)

      @pl.when(b == 1)
      def _():
        pltpu.async_copy(x_hbm.at[rows, pl.ds(col0, dh)], xv.at[1], semi1)
        pltpu.async_copy(b_hbm.at[rows], iv1, semi1)

    def in_wait(b):
      rows = pl.ds(0, _C)

      @pl.when(b == 0)
      def _():
        pltpu.make_async_copy(
            x_hbm.at[rows, pl.ds(0, dh)], xv.at[0], semi0).wait()
        pltpu.make_async_copy(b_hbm.at[rows], iv0, semi0).wait()

      @pl.when(b == 1)
      def _():
        pltpu.make_async_copy(
            x_hbm.at[rows, pl.ds(0, dh)], xv.at[1], semi1).wait()
        pltpu.make_async_copy(b_hbm.at[rows], iv1, semi1).wait()

    def out_start(c, b):
      rows = pl.ds(c * _C, _C)

      @pl.when(b == 0)
      def _():
        pltpu.async_copy(ov.at[0], out_hbm.at[rows, pl.ds(col0, dh)], semo0)

      @pl.when(b == 1)
      def _():
        pltpu.async_copy(ov.at[1], out_hbm.at[rows, pl.ds(col0, dh)], semo1)

    def out_wait(b):
      rows = pl.ds(0, _C)

      @pl.when(b == 0)
      def _():
        pltpu.make_async_copy(
            ov.at[0], out_hbm.at[rows, pl.ds(0, dh)], semo0).wait()

      @pl.when(b == 1)
      def _():
        pltpu.make_async_copy(
            ov.at[1], out_hbm.at[rows, pl.ds(0, dh)], semo1).wait()

    def gvec_of(b, q):
      return jnp.where(b == 0, iv0[pl.ds(q * _L, _L)],
                       iv1[pl.ds(q * _L, _L)])

    in_start(lo, 0)
    pltpu.sync_copy(a_hbm.at[pl.ds(col0, dh)], av)
    pltpu.sync_copy(w_hbm.at[pl.ds(col0, dh)], wv)
    pltpu.sync_copy(bias_hbm.at[pl.ds(col0, dh)], bv)

    def zero_body(g, carry):
      for f in range(nfh):
        s = pl.ds(f * _L, _L)
        sumv[g, s] = zeros
        sqv[g, s] = zeros
      cntv[g, :] = zeros
      return carry

    lax.fori_loop(0, _G, zero_body, 0)

    def stats_chunk(c, carry):
      b = lax.rem(c - lo, 2)

      @pl.when(c + 1 < hi)
      def _():
        in_start(c + 1, 1 - b)

      in_wait(b)

      cf = gvec_of(b, 0)[0]
      cl = gvec_of(b, _C // _L - 1)[_L - 1]

      @pl.when(cf == cl)
      def _uniform():
        def ugrp(q, accs):
          accs = list(accs)
          for j in range(_L):
            r = q * _L + j
            for f in range(nfh):
              s = pl.ds(f * _L, _L)
              v = xv[b, r, s]
              accs[f] = accs[f] + v
              accs[nfh + f] = accs[nfh + f] + v * v
          return tuple(accs)


        init = tuple([jnp.zeros((_L,), jnp.float32)] * (2 * nfh))
        accs = lax.fori_loop(0, _C // _L, ugrp, init)
        for f in range(nfh):
          s = pl.ds(f * _L, _L)
          plsc.addupdate(sumv.at[cf, s], accs[f])
          plsc.addupdate(sqv.at[cf, s], accs[nfh + f])
        plsc.addupdate(cntv.at[cf, :], ones * float(_C))

      @pl.when(cf != cl)
      def _mixed():
        mixed_groups(b)
      return carry

    def mixed_groups(b):
      def grp_body(q, rc):
        gvec = gvec_of(b, q)
        g0 = gvec[0]
        g15 = gvec[_L - 1]

        @pl.when(g0 == g15)
        def _fast():
          accs = []
          accq = []
          for f in range(nfh):
            s = pl.ds(f * _L, _L)
            v = xv[b, q * _L, s]
            accs.append(v)
            accq.append(v * v)
          for j in range(1, _L):
            r = q * _L + j
            for f in range(nfh):
              s = pl.ds(f * _L, _L)
              v = xv[b, r, s]
              accs[f] = accs[f] + v
              accq[f] = accq[f] + v * v
          for f in range(nfh):
            s = pl.ds(f * _L, _L)
            plsc.addupdate(sumv.at[g0, s], accs[f])
            plsc.addupdate(sqv.at[g0, s], accq[f])
          plsc.addupdate(cntv.at[g0, :], ones * float(_L))

        @pl.when(g0 != g15)
        def _slow():
          for j in range(_L):
            g = gvec[j]
            r = q * _L + j
            for f in range(nfh):
              s = pl.ds(f * _L, _L)
              v = xv[b, r, s]
              plsc.addupdate(sumv.at[g, s], v)
              plsc.addupdate(sqv.at[g, s], v * v)
            plsc.addupdate(cntv.at[g, :], ones)

        return rc

      lax.fori_loop(0, _C // _L, grp_body, 0)

    lax.fori_loop(lo, hi, stats_chunk, 0)

    g0s = sid * _GS

    in_start(lo, 0)

    @pl.when(lo + 1 < hi)
    def _():
      in_start(lo + 1, 1)

    pltpu.sync_copy(sumv, spm_p.at[sid])
    pltpu.sync_copy(cntv, spm_c.at[sid])
    plsc.subcore_barrier()

    handles = []
    for t in range(_NS):
      handles.append(pltpu.async_copy(
          spm_c.at[t, pl.ds(g0s, _GS)], red_c.at[t], semr))
    for h in handles:
      h.wait()

    handles = []
    for t in range(_NS):
      handles.append(pltpu.async_copy(
          spm_p.at[t, pl.ds(g0s, _GS)], red.at[t], semr))
    for h in handles:
      h.wait()

    def red_s_body(gi, carry):
      for f in range(nfh):
        s = pl.ds(f * _L, _L)
        acc = red[0, gi, s]
        for t in range(1, _NS):
          acc = acc + red[t, gi, s]
        acc_s[gi, s] = acc
      return carry

    lax.fori_loop(0, _GS, red_s_body, 0)

    plsc.subcore_barrier()
    pltpu.sync_copy(sqv, spm_p.at[sid])
    plsc.subcore_barrier()

    handles = []
    for t in range(_NS):
      handles.append(pltpu.async_copy(
          spm_p.at[t, pl.ds(g0s, _GS)], red.at[t], semr))
    for h in handles:
      h.wait()

    def red_q_body(gi, carry):
      for f in range(nfh):
        s = pl.ds(f * _L, _L)
        acc = red[0, gi, s]
        for t in range(1, _NS):
          acc = acc + red[t, gi, s]
        acc_q[gi, s] = acc
      return carry

    lax.fori_loop(0, _GS, red_q_body, 0)

    def scale_body(gi, carry):
      cvec = red_c[0, gi, :]
      for t in range(1, _NS):
        cvec = cvec + red_c[t, gi, :]
      cnt = jnp.maximum(cvec, 1.0)
      rcnt = 1.0 / cnt
      for f in range(nfh):
        s = pl.ds(f * _L, _L)
        mu = acc_s[gi, s] * rcnt
        m2 = acc_q[gi, s] * rcnt
        a = av[s]
        w = wv[s]
        bb = bv[s]
        sig2 = m2 - (2.0 * a - a * a) * mu * mu
        sig2 = jnp.maximum(sig2, 0.0) + _EPS
        y = _nr_rsqrt(sig2)
        slc_sc[gi, s] = w * y
        slc_sh[gi, s] = bb - w * a * mu * y
      return carry

    lax.fori_loop(0, _GS, scale_body, 0)

    pltpu.sync_copy(slc_sc, spm_sc.at[pl.ds(g0s, _GS)])
    pltpu.sync_copy(slc_sh, spm_sh.at[pl.ds(g0s, _GS)])
    plsc.subcore_barrier()

    def fetch(g):
      pltpu.sync_copy(spm_sc.at[pl.ds(g, 1)], stag_sc)
      pltpu.sync_copy(spm_sh.at[pl.ds(g, 1)], stag_sh)

    def apply_chunk(c, cur):
      b = lax.rem(c - lo, 2)

      @pl.when(c + 1 < hi)
      def _():
        in_start(c + 1, 1 - b)

      in_wait(b)

      @pl.when(c - 2 >= lo)
      def _():
        out_wait(b)

      cf = gvec_of(b, 0)[0]
      cl = gvec_of(b, _C // _L - 1)[_L - 1]

      @pl.when(cf != cur)
      def _():
        fetch(cf)

      def uni_grp(q, rc):
        scr = []
        shr = []
        for f in range(nfh):
          s = pl.ds(f * _L, _L)
          scr.append(stag_sc[0, s])
          shr.append(stag_sh[0, s])
        for j0 in range(0, _L, 4):
          vals = []
          for j in range(4):
            r = q * _L + j0 + j
            for f in range(nfh):
              vals.append(xv[b, r, pl.ds(f * _L, _L)])
          res = []
          for j in range(4):
            for f in range(nfh):
              res.append(vals[j * nfh + f] * scr[f] + shr[f])
          for j in range(4):
            r = q * _L + j0 + j
            for f in range(nfh):
              ov[b, r, pl.ds(f * _L, _L)] = res[j * nfh + f]
        return rc

      @pl.when(cf == cl)
      def _uniform():
        lax.fori_loop(0, _C // _L, uni_grp, 0)

      def grp_body(q, gcur):
        gvec = gvec_of(b, q)
        g0 = gvec[0]
        g15 = gvec[_L - 1]

        @pl.when(g0 != gcur)
        def _():
          fetch(g0)

        @pl.when(g0 == g15)
        def _fast():
          scr = []
          shr = []
          for f in range(nfh):
            s = pl.ds(f * _L, _L)
            scr.append(stag_sc[0, s])
            shr.append(stag_sh[0, s])
          for j0 in range(0, _L, 4):
            vals = []
            for j in range(4):
              r = q * _L + j0 + j
              for f in range(nfh):
                vals.append(xv[b, r, pl.ds(f * _L, _L)])
            res = []
            for j in range(4):
              for f in range(nfh):
                res.append(vals[j * nfh + f] * scr[f] + shr[f])
            for j in range(4):
              r = q * _L + j0 + j
              for f in range(nfh):
                ov[b, r, pl.ds(f * _L, _L)] = res[j * nfh + f]

        @pl.when(g0 != g15)
        def _slow():
          prev = g0
          for j in range(_L):
            g = gvec[j]
            r = q * _L + j
            if j > 0:
              pred = g != prev

              @pl.when(pred)
              def _():
                fetch(g)

            for f in range(nfh):
              s = pl.ds(f * _L, _L)
              ov[b, r, s] = xv[b, r, s] * stag_sc[0, s] + stag_sh[0, s]
            prev = g

        return g15

      @pl.when(cf != cl)
      def _mixed():
        lax.fori_loop(0, _C // _L, grp_body, cf)

      out_start(c, b)
      return cl

    lax.fori_loop(lo, hi, apply_chunk, jnp.int32(-1))

    def drain(i, carry):
      @pl.when(i >= lo)
      def _():
        out_wait(lax.rem(i - lo, 2))
      return carry

    lax.fori_loop(hi - 2, hi, drain, 0)

  return k(x, batch, alpha, weight, bias)


def kernel(x, batch, alpha, weight, bias):
  batch = batch.astype(jnp.int32)
  return _fused(x, batch, alpha, weight, bias)
